# baseline (speedup 1.0000x reference)
import functools

import jax
import jax.numpy as jnp
from jax.experimental import pallas as pl
from jax.experimental.pallas import tpu as pltpu

COULOMB = 322.0637


def _gather_body(A, bt_ref, n_ref, q_in, v_in, qm_out, v_out):
    p = pl.program_id(0)
    b = pl.program_id(1)
    n = n_ref[bt_ref[p, b]]
    lane = jax.lax.broadcasted_iota(jnp.int32, (1, 1, 1, A), 3)
    qm_out[...] = q_in[...] * (lane < n).astype(jnp.float32)
    v_out[...] = v_in[...]


def _dense_body(A, N, K, bt_ref, n_ref, gp, qcol_in, qrow_in, ucol_in, intra_in,
                xi_in, xiT_in, xT_in, qm_row_in, v_row_in, bexp_in, out_ref):
    p = pl.program_id(0)
    bi = pl.program_id(1)
    n = n_ref[bt_ref[p, bi]]
    J = N // K

    D = gp[0:1, 0:1]
    D0 = gp[0:1, 1:2]
    S = gp[0:1, 2:3]
    mind = gp[0:1, 3:4]
    maxd = gp[0:1, 4:5]

    def e_coul(x):
        xs = x * S
        eps = D - 0.5 * (D - D0) * (2.0 + 2.0 * xs + xs * xs) * jnp.exp(-xs)
        return COULOMB / (eps * x)

    e_shift = e_coul(maxd)

    xi = xi_in[0, 0]
    x0 = xi[:, 0:1]
    x1 = xi[:, 1:2]
    x2 = xi[:, 2:3]
    u = ucol_in[0]
    u0 = u[0]
    u1 = u[1]
    jstart = bi * A

    def chunk(j, acc):
        xT = xT_in[0, :, j, :]
        d2 = ((x0 - xT[0:1, :]) ** 2
              + (x1 - xT[1:2, :]) ** 2
              + (x2 - xT[2:3, :]) ** 2)
        d = jnp.sqrt(jnp.maximum(d2, 1.0))
        e = e_coul(jnp.clip(d, mind, maxd)) - e_shift

        v = v_row_in[0, :, j, :]
        bexp = bexp_in[0, 0, :, j, :]
        m0 = jnp.minimum(bexp[0:1] + v[0:1], bexp[1:2] + v[1:2])
        m1 = jnp.minimum(bexp[2:3] + v[0:1], bexp[3:4] + v[1:2])
        sep = jnp.minimum(u0 + m0, u1 + m1)
        w = jnp.clip(0.8 * sep.astype(jnp.float32) - 3.0, 0.0, 1.0)

        lane_j = j * K + jax.lax.broadcasted_iota(jnp.int32, (1, K), 1)
        notsame = (lane_j < jstart) | (lane_j >= jstart + A)
        row_eff = qm_row_in[0, 0:1, j, :] * notsame.astype(jnp.float32)

        return acc + e * (w * row_eff)

    acc = jnp.zeros((A, K), jnp.float32)
    for j in range(J):
        acc = chunk(j, acc)

    col_a = jax.lax.broadcasted_iota(jnp.int32, (A, 1), 0)
    qm_col = qcol_in[0] * (col_a < n).astype(jnp.float32)

    inter = jnp.sum(acc * qm_col, keepdims=True)

    xiT = xiT_in[0, 0]
    d2i = ((xi[:, 0:1] - xiT[0:1, :]) ** 2
           + (xi[:, 1:2] - xiT[1:2, :]) ** 2
           + (xi[:, 2:3] - xiT[2:3, :]) ** 2)
    di = jnp.sqrt(jnp.maximum(d2i, 1.0))
    ei = e_coul(jnp.clip(di, mind, maxd)) - e_shift
    sepi = intra_in[0]
    wi = jnp.clip(0.8 * sepi.astype(jnp.float32) - 3.0, 0.0, 1.0)
    row_a = jax.lax.broadcasted_iota(jnp.int32, (1, A), 1)
    qm_rowi = qrow_in[0] * (row_a < n).astype(jnp.float32)
    intra = jnp.sum(ei * wi * qm_rowi * qm_col, keepdims=True)

    part = 0.5 * (inter + intra)

    @pl.when(bi == 0)
    def _():
        out_ref[0] = jnp.zeros((1, 1), jnp.float32)

    out_ref[0] += part


def kernel(coords, pose_stack_block_coord_offset, pose_stack_block_types,
           pose_stack_min_block_bondsep, pose_stack_inter_block_bondsep,
           bt_n_atoms, bt_partial_charge, bt_n_interblock_bonds,
           bt_atoms_forming_chemical_bonds, bt_inter_repr_path_distance,
           bt_intra_repr_path_distance, global_params):
    P, B = pose_stack_block_types.shape
    T, A = bt_partial_charge.shape
    C = bt_inter_repr_path_distance.shape[1]
    N = B * A

    bt = pose_stack_block_types.astype(jnp.int32)
    n_at = bt_n_atoms.astype(jnp.int32)

    q_tab4 = bt_partial_charge[:, None, None, :]
    conn4 = bt_inter_repr_path_distance[:, None, :, :]
    grid_a = pltpu.PrefetchScalarGridSpec(
        num_scalar_prefetch=2,
        grid=(P, B),
        in_specs=[
            pl.BlockSpec((1, 1, 1, A), lambda p, b, btr, nr: (btr[p, b], 0, 0, 0)),
            pl.BlockSpec((1, 1, C, A), lambda p, b, btr, nr: (btr[p, b], 0, 0, 0)),
        ],
        out_specs=[
            pl.BlockSpec((1, 1, 1, A), lambda p, b, btr, nr: (p, b, 0, 0)),
            pl.BlockSpec((1, 1, C, A), lambda p, b, btr, nr: (p, b, 0, 0)),
        ],
    )
    qm_pb, v_pb = pl.pallas_call(
        functools.partial(_gather_body, A),
        grid_spec=grid_a,
        out_shape=[
            jax.ShapeDtypeStruct((P, B, 1, A), jnp.float32),
            jax.ShapeDtypeStruct((P, B, C, A), jnp.int32),
        ],
    )(bt, n_at, q_tab4, conn4)

    K = 128
    J = N // K
    qm_row4 = qm_pb.reshape(P, 1, J, K)
    v_row = v_pb.transpose(0, 2, 1, 3).reshape(P, C, J, K)
    ib = pose_stack_inter_block_bondsep.astype(jnp.int32)
    t = ib.transpose(0, 1, 3, 4, 2).reshape(P, B, C * C, B)
    b_exp = jnp.broadcast_to(t[..., None], (P, B, C * C, B, A)).reshape(
        P, B, C * C, J, K)

    coords_pb4 = coords.reshape(P, B, A, 3)
    coords_bT = coords_pb4.transpose(0, 1, 3, 2)
    coords_T = coords.transpose(0, 2, 1).reshape(P, 3, J, K)
    q_colT = bt_partial_charge[:, :, None]
    q_tab3 = bt_partial_charge[:, None, :]
    conn_colT = bt_inter_repr_path_distance[..., None]
    intra_tab = bt_intra_repr_path_distance.astype(jnp.int32)
    gp = global_params.astype(jnp.float32)

    grid_b = pltpu.PrefetchScalarGridSpec(
        num_scalar_prefetch=2,
        grid=(P, B),
        in_specs=[
            pl.BlockSpec((1, 5), lambda p, b, btr, nr: (0, 0)),
            pl.BlockSpec((1, A, 1), lambda p, b, btr, nr: (btr[p, b], 0, 0)),
            pl.BlockSpec((1, 1, A), lambda p, b, btr, nr: (btr[p, b], 0, 0)),
            pl.BlockSpec((1, C, A, 1), lambda p, b, btr, nr: (btr[p, b], 0, 0, 0)),
            pl.BlockSpec((1, A, A), lambda p, b, btr, nr: (btr[p, b], 0, 0)),
            pl.BlockSpec((1, 1, A, 3), lambda p, b, btr, nr: (p, b, 0, 0)),
            pl.BlockSpec((1, 1, 3, A), lambda p, b, btr, nr: (p, b, 0, 0)),
            pl.BlockSpec((1, 3, J, K), lambda p, b, btr, nr: (p, 0, 0, 0)),
            pl.BlockSpec((1, 1, J, K), lambda p, b, btr, nr: (p, 0, 0, 0)),
            pl.BlockSpec((1, C, J, K), lambda p, b, btr, nr: (p, 0, 0, 0)),
            pl.BlockSpec((1, 1, C * C, J, K),
                         lambda p, b, btr, nr: (p, b, 0, 0, 0)),
        ],
        out_specs=pl.BlockSpec((1, 1, 1), lambda p, b, btr, nr: (p, 0, 0)),
    )
    out3 = pl.pallas_call(
        functools.partial(_dense_body, A, N, K),
        grid_spec=grid_b,
        out_shape=jax.ShapeDtypeStruct((P, 1, 1), jnp.float32),
        compiler_params=pltpu.CompilerParams(
            dimension_semantics=("parallel", "arbitrary")),
    )(bt, n_at, gp, q_colT, q_tab3, conn_colT, intra_tab,
      coords_pb4, coords_bT, coords_T, qm_row4, v_row, b_exp)

    return out3.reshape(P)


# speedup vs baseline: 1.4640x; 1.4640x over previous
import functools

import jax
import jax.numpy as jnp
from jax.experimental import pallas as pl
from jax.experimental.pallas import tpu as pltpu

COULOMB = 322.0637


def _gather_body(A, bt_ref, n_ref, q_in, v_in, qm_out, v_out):
    p = pl.program_id(0)
    b = pl.program_id(1)
    n = n_ref[bt_ref[p, b]]
    lane = jax.lax.broadcasted_iota(jnp.int32, (1, 1, 1, A), 3)
    qm_out[...] = q_in[...] * (lane < n).astype(jnp.float32)
    v_out[...] = v_in[...]


def _dense_body(A, N, K, bt_ref, n_ref, gp, qcol_in, qrow_in, ucol_in, intra_in,
                xi_in, xiT_in, xT_in, qm_row_in, v_row_in, bexp_in, out_ref):
    p = pl.program_id(0)
    bi = pl.program_id(1)
    n = n_ref[bt_ref[p, bi]]
    J = N // K

    D = gp[0:1, 0:1]
    D0 = gp[0:1, 1:2]
    S = gp[0:1, 2:3]
    mind = gp[0:1, 3:4]
    maxd = gp[0:1, 4:5]

    def e_coul(x):
        xs = x * S
        eps = D - 0.5 * (D - D0) * (2.0 + 2.0 * xs + xs * xs) * jnp.exp(-xs)
        return COULOMB / (eps * x)

    e_shift = e_coul(maxd)

    xi = xi_in[0, 0]
    x0 = xi[:, 0:1]
    x1 = xi[:, 1:2]
    x2 = xi[:, 2:3]
    u = ucol_in[0]
    u0 = u[0]
    u1 = u[1]
    jstart = bi * A

    def chunk(j, acc):
        xT = xT_in[0, :, j, :]
        d2 = ((x0 - xT[0:1, :]) ** 2
              + (x1 - xT[1:2, :]) ** 2
              + (x2 - xT[2:3, :]) ** 2)
        d = jnp.sqrt(jnp.maximum(d2, 1.0))
        e = e_coul(jnp.clip(d, mind, maxd)) - e_shift

        v = v_row_in[0, :, j, :]
        bexp = bexp_in[0, 0, :, j, :]
        m0 = jnp.minimum(bexp[0:1] + v[0:1], bexp[1:2] + v[1:2])
        m1 = jnp.minimum(bexp[2:3] + v[0:1], bexp[3:4] + v[1:2])
        sep = jnp.minimum(u0 + m0, u1 + m1)
        w = jnp.clip(0.8 * sep.astype(jnp.float32) - 3.0, 0.0, 1.0)

        lane_j = j * K + jax.lax.broadcasted_iota(jnp.int32, (1, K), 1)
        notsame = (lane_j < jstart) | (lane_j >= jstart + A)
        row_eff = qm_row_in[0, 0:1, j, :] * notsame.astype(jnp.float32)

        return acc + e * (w * row_eff)

    acc = jnp.zeros((A, K), jnp.float32)
    for j in range(J):
        acc = chunk(j, acc)

    col_a = jax.lax.broadcasted_iota(jnp.int32, (A, 1), 0)
    qm_col = qcol_in[0] * (col_a < n).astype(jnp.float32)

    inter = jnp.sum(acc * qm_col, keepdims=True)

    xiT = xiT_in[0, 0]
    d2i = ((xi[:, 0:1] - xiT[0:1, :]) ** 2
           + (xi[:, 1:2] - xiT[1:2, :]) ** 2
           + (xi[:, 2:3] - xiT[2:3, :]) ** 2)
    di = jnp.sqrt(jnp.maximum(d2i, 1.0))
    ei = e_coul(jnp.clip(di, mind, maxd)) - e_shift
    sepi = intra_in[0]
    wi = jnp.clip(0.8 * sepi.astype(jnp.float32) - 3.0, 0.0, 1.0)
    row_a = jax.lax.broadcasted_iota(jnp.int32, (1, A), 1)
    qm_rowi = qrow_in[0] * (row_a < n).astype(jnp.float32)
    intra = jnp.sum(ei * wi * qm_rowi * qm_col, keepdims=True)

    part = 0.5 * (inter + intra)

    @pl.when(bi == 0)
    def _():
        out_ref[0] = jnp.zeros((1, 1), jnp.float32)

    out_ref[0] += part


def kernel(coords, pose_stack_block_coord_offset, pose_stack_block_types,
           pose_stack_min_block_bondsep, pose_stack_inter_block_bondsep,
           bt_n_atoms, bt_partial_charge, bt_n_interblock_bonds,
           bt_atoms_forming_chemical_bonds, bt_inter_repr_path_distance,
           bt_intra_repr_path_distance, global_params):
    P, B = pose_stack_block_types.shape
    T, A = bt_partial_charge.shape
    C = bt_inter_repr_path_distance.shape[1]
    N = B * A

    bt = pose_stack_block_types.astype(jnp.int32)
    n_at = bt_n_atoms.astype(jnp.int32)

    q_tab4 = bt_partial_charge[:, None, None, :]
    conn4 = bt_inter_repr_path_distance[:, None, :, :]
    grid_a = pltpu.PrefetchScalarGridSpec(
        num_scalar_prefetch=2,
        grid=(P, B),
        in_specs=[
            pl.BlockSpec((1, 1, 1, A), lambda p, b, btr, nr: (btr[p, b], 0, 0, 0)),
            pl.BlockSpec((1, 1, C, A), lambda p, b, btr, nr: (btr[p, b], 0, 0, 0)),
        ],
        out_specs=[
            pl.BlockSpec((1, 1, 1, A), lambda p, b, btr, nr: (p, b, 0, 0)),
            pl.BlockSpec((1, 1, C, A), lambda p, b, btr, nr: (p, b, 0, 0)),
        ],
    )
    qmask_tab = bt_partial_charge * (
        jnp.arange(A)[None, :] < n_at[:, None]).astype(jnp.float32)
    qm_pb = jnp.take(qmask_tab, bt, axis=0)[:, :, None, :]
    v_pb = jnp.take(bt_inter_repr_path_distance.astype(jnp.int32), bt, axis=0)

    K = 128
    J = N // K
    qm_row4 = qm_pb.reshape(P, 1, J, K)
    v_row = v_pb.transpose(0, 2, 1, 3).reshape(P, C, J, K)
    ib = pose_stack_inter_block_bondsep.astype(jnp.int32)
    t = ib.transpose(0, 1, 3, 4, 2).reshape(P, B, C * C, B)
    b_exp = jnp.broadcast_to(t[..., None], (P, B, C * C, B, A)).reshape(
        P, B, C * C, J, K)

    coords_pb4 = coords.reshape(P, B, A, 3)
    coords_bT = coords_pb4.transpose(0, 1, 3, 2)
    coords_T = coords.transpose(0, 2, 1).reshape(P, 3, J, K)
    q_colT = bt_partial_charge[:, :, None]
    q_tab3 = bt_partial_charge[:, None, :]
    conn_colT = bt_inter_repr_path_distance[..., None]
    intra_tab = bt_intra_repr_path_distance.astype(jnp.int32)
    gp = global_params.astype(jnp.float32)

    grid_b = pltpu.PrefetchScalarGridSpec(
        num_scalar_prefetch=2,
        grid=(P, B),
        in_specs=[
            pl.BlockSpec((1, 5), lambda p, b, btr, nr: (0, 0)),
            pl.BlockSpec((1, A, 1), lambda p, b, btr, nr: (btr[p, b], 0, 0)),
            pl.BlockSpec((1, 1, A), lambda p, b, btr, nr: (btr[p, b], 0, 0)),
            pl.BlockSpec((1, C, A, 1), lambda p, b, btr, nr: (btr[p, b], 0, 0, 0)),
            pl.BlockSpec((1, A, A), lambda p, b, btr, nr: (btr[p, b], 0, 0)),
            pl.BlockSpec((1, 1, A, 3), lambda p, b, btr, nr: (p, b, 0, 0)),
            pl.BlockSpec((1, 1, 3, A), lambda p, b, btr, nr: (p, b, 0, 0)),
            pl.BlockSpec((1, 3, J, K), lambda p, b, btr, nr: (p, 0, 0, 0)),
            pl.BlockSpec((1, 1, J, K), lambda p, b, btr, nr: (p, 0, 0, 0)),
            pl.BlockSpec((1, C, J, K), lambda p, b, btr, nr: (p, 0, 0, 0)),
            pl.BlockSpec((1, 1, C * C, J, K),
                         lambda p, b, btr, nr: (p, b, 0, 0, 0)),
        ],
        out_specs=pl.BlockSpec((1, 1, 1), lambda p, b, btr, nr: (p, 0, 0)),
    )
    out3 = pl.pallas_call(
        functools.partial(_dense_body, A, N, K),
        grid_spec=grid_b,
        out_shape=jax.ShapeDtypeStruct((P, 1, 1), jnp.float32),
        compiler_params=pltpu.CompilerParams(
            dimension_semantics=("parallel", "arbitrary")),
    )(bt, n_at, gp, q_colT, q_tab3, conn_colT, intra_tab,
      coords_pb4, coords_bT, coords_T, qm_row4, v_row, b_exp)

    return out3.reshape(P)
